# baseline (speedup 1.0000x reference)
import functools

import jax
import jax.numpy as jnp
from jax import lax
from jax.experimental import pallas as pl

_MASK_PROB = 0.15

_XBLK = 2000
_EBLK = 8000


def _masks():
    key = jax.random.key(42)
    kn, ke = jax.random.split(key)
    node_mask = jax.random.uniform(kn, (10000,)) < _MASK_PROB
    edge_mask = jax.random.uniform(ke, (320000,)) < _MASK_PROB
    node_keep = (1.0 - node_mask.astype(jnp.float32)).reshape(10000, 1)
    ek = 1.0 - edge_mask.astype(jnp.float32)
    edge_keep = ek.reshape(5000, 8, 8).transpose(0, 2, 1).reshape(40000, 8)
    return node_keep, edge_keep


def _body(nx_blocks, nm_ref, em_ref, x_ref, e_ref, ox_ref, oe_ref):
    i = pl.program_id(0)

    @pl.when(i < nx_blocks)
    def _():
        ox_ref[...] = x_ref[...] * nm_ref[...]

    @pl.when(i >= nx_blocks)
    def _():
        row = lax.broadcasted_iota(jnp.int32, (8, 128), 0)
        lane = lax.broadcasted_iota(jnp.int32, (8, 128), 1)
        expand = (lane // 16 == row).astype(jnp.float32)
        keep = lax.dot(em_ref[...], expand, preferred_element_type=jnp.float32)
        oe_ref[...] = e_ref[...] * keep


def kernel(x, edge_attr):
    n_nodes, dx = x.shape
    n_edges, de = edge_attr.shape
    node_keep, edge_keep = _masks()

    e128 = (edge_attr.reshape(5000, 8, 8, de)
            .transpose(0, 2, 1, 3)
            .reshape(n_edges * de // 128, 128))
    one = lax.optimization_barrier(jnp.float32(1.0))
    e128 = e128 * one
    n_erows = e128.shape[0]

    nx_blocks = n_nodes // _XBLK
    ne_blocks = n_erows // _EBLK
    grid = nx_blocks + ne_blocks

    def x_map(i):
        return (jnp.minimum(i, nx_blocks - 1), 0)

    def e_map(i):
        return (jnp.maximum(i - nx_blocks, 0), 0)

    body = functools.partial(_body, nx_blocks)

    ox, oe = pl.pallas_call(
        body,
        grid=(grid,),
        in_specs=[
            pl.BlockSpec((_XBLK, 1), x_map),
            pl.BlockSpec((_EBLK, 8), e_map),
            pl.BlockSpec((_XBLK, dx), x_map),
            pl.BlockSpec((_EBLK, 128), e_map),
        ],
        out_specs=[
            pl.BlockSpec((_XBLK, dx), x_map),
            pl.BlockSpec((_EBLK, 128), e_map),
        ],
        out_shape=[
            jax.ShapeDtypeStruct((n_nodes, dx), x.dtype),
            jax.ShapeDtypeStruct((n_erows, 128), edge_attr.dtype),
        ],
    )(node_keep, edge_keep, x, e128)

    oe = (oe.reshape(5000, 8, 8, de)
          .transpose(0, 2, 1, 3)
          .reshape(n_edges, de))
    oe = oe * one
    return ox, oe


# speedup vs baseline: 1.5444x; 1.5444x over previous
import functools

import jax
import jax.numpy as jnp
from jax import lax
from jax.experimental import pallas as pl

_MASK_PROB = 0.15

_XBLK = 2000
_EBLK = 8000


def _masks():
    key = jax.random.key(42)
    kn, ke = jax.random.split(key)
    node_mask = jax.random.uniform(kn, (10000,)) < _MASK_PROB
    edge_mask = jax.random.uniform(ke, (320000,)) < _MASK_PROB
    node_keep = 1.0 - node_mask.astype(jnp.float32)
    edge_keep = 1.0 - edge_mask.astype(jnp.float32)
    nk = jnp.broadcast_to(node_keep[:, None], (10000, 128))
    ek = jnp.repeat(edge_keep.reshape(40000, 8), 16, axis=1)
    return nk, ek


def _body(nx_blocks, nm_ref, em_ref, x_ref, e_ref, ox_ref, oe_ref):
    i = pl.program_id(0)

    @pl.when(i < nx_blocks)
    def _():
        ox_ref[...] = x_ref[...] * nm_ref[...]

    @pl.when(i >= nx_blocks)
    def _():
        oe_ref[...] = e_ref[...] * em_ref[...]


def kernel(x, edge_attr):
    n_nodes, dx = x.shape
    n_edges, de = edge_attr.shape
    node_keep, edge_keep = _masks()

    e128 = edge_attr.reshape(n_edges * de // 128, 128)
    n_erows = e128.shape[0]

    nx_blocks = n_nodes // _XBLK
    ne_blocks = n_erows // _EBLK
    grid = nx_blocks + ne_blocks

    def x_map(i):
        return (jnp.minimum(i, nx_blocks - 1), 0)

    def e_map(i):
        return (jnp.maximum(i - nx_blocks, 0), 0)

    body = functools.partial(_body, nx_blocks)

    ox, oe = pl.pallas_call(
        body,
        grid=(grid,),
        in_specs=[
            pl.BlockSpec((_XBLK, dx), x_map),
            pl.BlockSpec((_EBLK, 128), e_map),
            pl.BlockSpec((_XBLK, dx), x_map),
            pl.BlockSpec((_EBLK, 128), e_map),
        ],
        out_specs=[
            pl.BlockSpec((_XBLK, dx), x_map),
            pl.BlockSpec((_EBLK, 128), e_map),
        ],
        out_shape=[
            jax.ShapeDtypeStruct((n_nodes, dx), x.dtype),
            jax.ShapeDtypeStruct((n_erows, 128), edge_attr.dtype),
        ],
    )(node_keep, edge_keep, x, e128)

    return ox, oe.reshape(n_edges, de)


# speedup vs baseline: 1.5933x; 1.0316x over previous
import functools

import jax
import jax.numpy as jnp
from jax import lax
from jax.experimental import pallas as pl

_MASK_PROB = 0.15

_XBLK = 2000
_EBLK = 8000


def _masks():
    key = jax.random.key(42)
    kn, ke = jax.random.split(key)
    node_mask = jax.random.uniform(kn, (10000,)) < _MASK_PROB
    edge_mask = jax.random.uniform(ke, (320000,)) < _MASK_PROB
    node_keep = 1.0 - node_mask.astype(jnp.float32)
    edge_keep = 1.0 - edge_mask.astype(jnp.float32)
    nk = jnp.broadcast_to(node_keep[:, None], (10000, 128))
    ek = jnp.repeat(edge_keep.reshape(40000, 8), 16, axis=1)
    return nk, ek


def _mul_body(m_ref, a_ref, o_ref):
    o_ref[...] = a_ref[...] * m_ref[...]


def _masked(mask, arr, blk):
    n, d = arr.shape
    return pl.pallas_call(
        _mul_body,
        grid=(n // blk,),
        in_specs=[
            pl.BlockSpec((blk, d), lambda i: (i, 0)),
            pl.BlockSpec((blk, d), lambda i: (i, 0)),
        ],
        out_specs=pl.BlockSpec((blk, d), lambda i: (i, 0)),
        out_shape=jax.ShapeDtypeStruct((n, d), arr.dtype),
    )(mask, arr)


def kernel(x, edge_attr):
    n_nodes, dx = x.shape
    n_edges, de = edge_attr.shape
    node_keep, edge_keep = _masks()

    e128 = edge_attr.reshape(n_edges * de // 128, 128)

    ox = _masked(node_keep, x, _XBLK)
    oe = _masked(edge_keep, e128, _EBLK)

    return ox, oe.reshape(n_edges, de)
